# baseline (speedup 1.0000x reference)
import functools

import jax
import jax.numpy as jnp
from jax import lax
from jax.experimental import pallas as pl
from jax.experimental.pallas import tpu as pltpu
from jax.experimental.pallas import tpu_sc as plsc

D = 2048
N = 8192
NW = 32
CPW = N // NW
L = 16
G = CPW // L
R = 128
NCH = D // R

_MESH = plsc.VectorSubcoreMesh(core_axis_name="c", subcore_axis_name="s")

_GATHER_DNUMS = lax.GatherDimensionNumbers(
    offset_dims=(), collapsed_slice_dims=(0,), start_index_map=(0,))


def _permute(x, perm):
    return lax.gather(x, perm[:, None], _GATHER_DNUMS, slice_sizes=(1,),
                      mode=lax.GatherScatterMode.PROMISE_IN_BOUNDS)


@functools.partial(
    pl.kernel,
    out_type=(
        jax.ShapeDtypeStruct((NW, L), jnp.float32),
        jax.ShapeDtypeStruct((NW, L), jnp.int32),
    ),
    mesh=_MESH,
    scratch_types=[
        pltpu.VMEM((D,), jnp.float32),
        pltpu.VMEM((R, CPW), jnp.float32),
        pltpu.VMEM((R, CPW), jnp.float32),
        pltpu.VMEM((L,), jnp.float32),
        pltpu.VMEM((L,), jnp.int32),
        pltpu.SemaphoreType.DMA,
        pltpu.SemaphoreType.DMA,
    ],
)
def _k1(x_hbm, w_hbm, oval, oidx, x_v, buf0, buf1, val_v, idx_v, sem0, sem1):
    wid = lax.axis_index("c") * 16 + lax.axis_index("s")
    col0 = wid * CPW
    pltpu.sync_copy(x_hbm, x_v)
    bufs = (buf0, buf1)
    sems = (sem0, sem1)
    pltpu.async_copy(w_hbm.at[pl.ds(0, R), pl.ds(col0, CPW)], buf0, sem0)
    pltpu.async_copy(w_hbm.at[pl.ds(R, R), pl.ds(col0, CPW)], buf1, sem1)

    def pair_body(p, acc):
        for b in range(2):
            c = 2 * p + b
            pltpu.make_async_copy(
                w_hbm.at[pl.ds(0, R), pl.ds(0, CPW)], bufs[b], sems[b]).wait()

            def blk_body(k, a, _buf=bufs[b], _c=c):
                xv = x_v[pl.ds(_c * R + k * L, L)]
                for j in range(L):
                    xi = xv[j]
                    a = tuple(a[g] + xi * _buf[k * L + j, pl.ds(g * L, L)]
                              for g in range(G))
                return a

            acc = lax.fori_loop(0, R // L, blk_body, acc)

            @pl.when(c + 2 < NCH)
            def _start_next(_b=b, _c=c):
                pltpu.async_copy(
                    w_hbm.at[pl.ds((_c + 2) * R, R), pl.ds(col0, CPW)],
                    bufs[_b], sems[_b])
        return acc

    acc = lax.fori_loop(
        0, NCH // 2, pair_body,
        tuple(jnp.zeros((L,), jnp.float32) for _ in range(G)))

    lanes = lax.iota(jnp.int32, L)
    mval = acc[0]
    midx = lanes + col0
    for g in range(1, G):
        better = acc[g] > mval
        mval = jnp.where(better, acc[g], mval)
        midx = jnp.where(better, lanes + (col0 + g * L), midx)
    val_v[...] = mval
    idx_v[...] = midx
    pltpu.sync_copy(val_v, oval.at[wid])
    pltpu.sync_copy(idx_v, oidx.at[wid])


@functools.partial(
    pl.kernel,
    out_type=jax.ShapeDtypeStruct((N,), jnp.float32),
    mesh=_MESH,
    scratch_types=[
        pltpu.VMEM((NW, L), jnp.float32),
        pltpu.VMEM((NW, L), jnp.int32),
        pltpu.VMEM((CPW,), jnp.float32),
    ],
)
def _k2(vals_hbm, idxs_hbm, out_hbm, val_v, idx_v, out_v):
    wid = lax.axis_index("c") * 16 + lax.axis_index("s")
    col0 = wid * CPW
    pltpu.sync_copy(vals_hbm, val_v)
    pltpu.sync_copy(idxs_hbm, idx_v)
    mval = val_v[0, :]
    midx = idx_v[0, :]
    for s in range(1, NW):
        v = val_v[s, :]
        ix = idx_v[s, :]
        better = v > mval
        mval = jnp.where(better, v, mval)
        midx = jnp.where(better, ix, midx)
    lanes = lax.iota(jnp.int32, L)
    for shift in (8, 4, 2, 1):
        perm = lanes ^ shift
        pv = _permute(mval, perm)
        pix = _permute(midx, perm)
        better = (pv > mval) | ((pv == mval) & (pix < midx))
        mval = jnp.where(better, pv, mval)
        midx = jnp.where(better, pix, midx)
    winner = midx[0]
    one = jnp.full((L,), 1.0, jnp.float32)
    zero = jnp.zeros((L,), jnp.float32)
    for g in range(G):
        out_v[pl.ds(g * L, L)] = jnp.where(
            lanes + (col0 + g * L) == winner, one, zero)
    pltpu.sync_copy(out_v, out_hbm.at[pl.ds(col0, CPW)])


def kernel(input_spikes, weights):
    vals, idxs = _k1(input_spikes, weights)
    return _k2(vals, idxs)


# speedup vs baseline: 1.6119x; 1.6119x over previous
import functools

import jax
import jax.numpy as jnp
from jax import lax
from jax.experimental import pallas as pl
from jax.experimental.pallas import tpu as pltpu
from jax.experimental.pallas import tpu_sc as plsc

D = 2048
N = 8192
L = 16

NTC = 4608
TCB = 512
NB = NTC // TCB

NSC = N - NTC
NW = 32
CPW = 128
NACT = NSC // CPW
G = CPW // L
R = 256
NCH = D // R

_BIG = 2**31 - 1

_MESH = plsc.VectorSubcoreMesh(core_axis_name="c", subcore_axis_name="s")


@functools.partial(
    pl.kernel,
    out_type=(
        jax.ShapeDtypeStruct((NACT, L), jnp.float32),
        jax.ShapeDtypeStruct((NACT, L), jnp.int32),
    ),
    mesh=_MESH,
    scratch_types=[
        pltpu.VMEM((D,), jnp.float32),
        pltpu.VMEM((R, CPW), jnp.float32),
        pltpu.VMEM((R, CPW), jnp.float32),
        pltpu.VMEM((L,), jnp.float32),
        pltpu.VMEM((L,), jnp.int32),
        pltpu.SemaphoreType.DMA,
        pltpu.SemaphoreType.DMA,
    ],
)
def _sc_partial(x_hbm, w_hbm, oval, oidx, x_v, buf0, buf1, val_v, idx_v,
                sem0, sem1):
    wid = lax.axis_index("s") * 2 + lax.axis_index("c")
    col0 = NTC + wid * CPW

    @pl.when(wid < NACT)
    def _active():
        _sc_worker(wid, col0, x_hbm, w_hbm, oval, oidx, x_v, buf0, buf1,
                   val_v, idx_v, sem0, sem1)


def _sc_worker(wid, col0, x_hbm, w_hbm, oval, oidx, x_v, buf0, buf1,
               val_v, idx_v, sem0, sem1):
    pltpu.sync_copy(x_hbm, x_v)
    bufs = (buf0, buf1)
    sems = (sem0, sem1)
    pltpu.async_copy(w_hbm.at[pl.ds(0, R), pl.ds(col0, CPW)], buf0, sem0)
    pltpu.async_copy(w_hbm.at[pl.ds(R, R), pl.ds(col0, CPW)], buf1, sem1)

    def pair_body(p, acc):
        for b in range(2):
            c = 2 * p + b
            pltpu.make_async_copy(
                w_hbm.at[pl.ds(0, R), pl.ds(0, CPW)], bufs[b], sems[b]).wait()

            def blk_body(k, a, _buf=bufs[b], _c=c):
                xv = x_v[pl.ds(_c * R + k * L, L)]
                xb = [xv[j] for j in range(L)]
                out = []
                for g in range(G):
                    a0 = a[g]
                    a1 = xb[0] * _buf[k * L, pl.ds(g * L, L)]
                    for j in range(1, L, 2):
                        a0 = a0 + xb[j] * _buf[k * L + j, pl.ds(g * L, L)]
                        if j + 1 < L:
                            a1 = a1 + xb[j + 1] * _buf[k * L + j + 1,
                                                       pl.ds(g * L, L)]
                    out.append(a0 + a1)
                return tuple(out)

            acc = lax.fori_loop(0, R // L, blk_body, acc)

            @pl.when(c + 2 < NCH)
            def _start_next(_b=b, _c=c):
                pltpu.async_copy(
                    w_hbm.at[pl.ds((_c + 2) * R, R), pl.ds(col0, CPW)],
                    bufs[_b], sems[_b])
        return acc

    acc = lax.fori_loop(
        0, NCH // 2, pair_body,
        tuple(jnp.zeros((L,), jnp.float32) for _ in range(G)))

    lanes = lax.iota(jnp.int32, L)
    mval = acc[0]
    midx = lanes + col0
    for g in range(1, G):
        better = acc[g] > mval
        mval = jnp.where(better, acc[g], mval)
        midx = jnp.where(better, lanes + (col0 + g * L), midx)
    val_v[...] = mval
    idx_v[...] = midx
    pltpu.sync_copy(val_v, oval.at[wid])
    pltpu.sync_copy(idx_v, oidx.at[wid])


def _tc_body(x_ref, w_ref, val_ref, idx_ref):
    j = pl.program_id(0)
    act = jnp.dot(x_ref[...], w_ref[...],
                  preferred_element_type=jnp.float32)
    m = jnp.max(act)
    cols = j * TCB + lax.broadcasted_iota(jnp.int32, (1, TCB), 1)
    am = jnp.min(jnp.where(act == m, cols, _BIG))
    val_ref[0, 0, 0] = m
    idx_ref[0, 0, 0] = am


_tc_partial = pl.pallas_call(
    _tc_body,
    grid=(NB,),
    in_specs=[
        pl.BlockSpec((1, D), lambda j: (0, 0)),
        pl.BlockSpec((D, TCB), lambda j: (0, j)),
    ],
    out_specs=[
        pl.BlockSpec((1, 1, 1), lambda j: (j, 0, 0), memory_space=pltpu.SMEM),
        pl.BlockSpec((1, 1, 1), lambda j: (j, 0, 0), memory_space=pltpu.SMEM),
    ],
    out_shape=[
        jax.ShapeDtypeStruct((NB, 1, 1), jnp.float32),
        jax.ShapeDtypeStruct((NB, 1, 1), jnp.int32),
    ],
)


def _merge_body(scv_ref, sci_ref, tcv_ref, tci_ref, out_ref):
    scv = scv_ref[...]
    sci = sci_ref[...]
    tcv = tcv_ref[...]
    tci = tci_ref[...]
    m = jnp.maximum(jnp.max(scv), jnp.max(tcv))
    w_sc = jnp.min(jnp.where(scv == m, sci, _BIG))
    w_tc = jnp.min(jnp.where(tcv == m, tci, _BIG))
    winner = jnp.minimum(w_sc, w_tc)
    flat = (lax.broadcasted_iota(jnp.int32, (64, 128), 0) * 128
            + lax.broadcasted_iota(jnp.int32, (64, 128), 1))
    out_ref[...] = jnp.where(flat == winner, jnp.float32(1.0),
                             jnp.float32(0.0))


_merge = pl.pallas_call(
    _merge_body,
    out_shape=jax.ShapeDtypeStruct((64, 128), jnp.float32),
)


def kernel(input_spikes, weights):
    tcv, tci = _tc_partial(input_spikes.reshape(1, D), weights)
    scv, sci = _sc_partial(input_spikes, weights)
    out2d = _merge(scv, sci, tcv, tci)
    return out2d.reshape(N)
